# Initial kernel scaffold; baseline (speedup 1.0000x reference)
import jax
import jax.numpy as jnp
from jax.experimental import pallas as pl


def kernel(inputs):
    raise NotImplementedError("write your pallas kernel here")


# speedup vs baseline: 10.1623x; 10.1623x over previous
import jax
import jax.numpy as jnp
from jax.experimental import pallas as pl

K_NN = 20
THRES = 0.5
P = 4096
ROWS_PER_BLOCK = 128
OUT_PAD = 128


def _topk_body(x_ref, o_ref):
    x = x_ref[...]
    R = x.shape[0]
    lane = jax.lax.broadcasted_iota(jnp.int32, (R, P), 1)
    base = pl.program_id(0) * ROWS_PER_BLOCK
    self_idx = (base % P) + jax.lax.broadcasted_iota(jnp.int32, (R, 1), 0)
    out_lane = jax.lax.broadcasted_iota(jnp.int32, (R, OUT_PAD), 1)
    acc = jnp.zeros((R, OUT_PAD), dtype=jnp.int32)
    big_i = jnp.int32(P)
    inf = jnp.float32(jnp.inf)
    for k in range(K_NN):
        m = jnp.min(x, axis=1, keepdims=True)
        is_min = x == m
        idx = jnp.min(jnp.where(is_min, lane, big_i), axis=1, keepdims=True)
        keep = m <= jnp.float32(THRES)
        sel = jnp.where(keep, idx, self_idx)
        acc = jnp.where(out_lane == k, sel, acc)
        x = jnp.where(lane == idx, inf, x)
    o_ref[...] = acc


def kernel(inputs):
    B, PP, _ = inputs.shape
    rows = B * PP
    x2 = inputs.reshape(rows, P)
    grid = (rows // ROWS_PER_BLOCK,)
    out = pl.pallas_call(
        _topk_body,
        grid=grid,
        in_specs=[pl.BlockSpec((ROWS_PER_BLOCK, P), lambda i: (i, 0))],
        out_specs=pl.BlockSpec((ROWS_PER_BLOCK, OUT_PAD), lambda i: (i, 0)),
        out_shape=jax.ShapeDtypeStruct((rows, OUT_PAD), jnp.int32),
    )(x2)
    return out[:, :K_NN].reshape(B, PP, K_NN)
